# baseline (speedup 1.0000x reference)
import functools

import jax
import jax.numpy as jnp
from jax import lax
from jax.experimental import pallas as pl
from jax.experimental.pallas import tpu as pltpu
from jax.experimental.pallas import tpu_sc as plsc

N = 10000
E = 320000
D = 128
EPS = 1e-5

NCORES = 2
NSUB = 16
CH = 64
CPC = 40
PHT = 8
F0 = 2
DCH = 128
DCT = PHT * CPC * CH // DCH
E_PAD = NSUB * PHT * CPC * CH
NP = 10240
RPT = NP // NSUB
PAD_ROW = 10008

_MESH = plsc.VectorSubcoreMesh(
    core_axis_name="c", subcore_axis_name="s",
    num_cores=NCORES, num_subcores=NSUB)


def _deg_body(srcs_hbm, dsts_hbm, zdeg_hbm, ones_hbm,
              do_hbm, di_hbm, h_out, h_in, sidx, didx, ones_v, sem_o, sem_i):
    c = lax.axis_index("c")
    s = lax.axis_index("s")
    pltpu.sync_copy(srcs_hbm.at[s], sidx)
    pltpu.sync_copy(dsts_hbm.at[s], didx)
    pltpu.sync_copy(ones_hbm, ones_v)
    sl = pl.ds(s * RPT, RPT)
    pltpu.sync_copy(zdeg_hbm.at[sl], h_out.at[sl])
    pltpu.sync_copy(zdeg_hbm.at[sl], h_in.at[sl])
    plsc.subcore_barrier()

    LAG = 4
    lo = c * (DCT // NCORES)
    hi = lo + DCT // NCORES

    @pl.loop(lo, hi)
    def _(j):
        pltpu.async_copy(ones_v, h_out.at[sidx.at[j]], sem_o, add=True)
        pltpu.async_copy(ones_v, h_in.at[didx.at[j]], sem_i, add=True)

        @pl.when(j >= lo + LAG)
        def _():
            pltpu.make_async_copy(ones_v, h_out.at[sidx.at[0]], sem_o).wait()
            pltpu.make_async_copy(ones_v, h_in.at[didx.at[0]], sem_i).wait()

    for _ in range(LAG):
        pltpu.make_async_copy(ones_v, h_out.at[sidx.at[0]], sem_o).wait()
        pltpu.make_async_copy(ones_v, h_in.at[didx.at[0]], sem_i).wait()

    plsc.subcore_barrier()
    pltpu.sync_copy(h_out.at[sl], do_hbm.at[c, sl])
    pltpu.sync_copy(h_in.at[sl], di_hbm.at[c, sl])


_deg_call = pl.kernel(
    _deg_body,
    out_type=[jax.ShapeDtypeStruct((NCORES, NP, 16), jnp.float32),
              jax.ShapeDtypeStruct((NCORES, NP, 16), jnp.float32)],
    mesh=_MESH,
    scratch_types=[
        pltpu.VMEM_SHARED((NP, 16), jnp.float32),
        pltpu.VMEM_SHARED((NP, 16), jnp.float32),
        pltpu.VMEM((DCT, DCH), jnp.int32),
        pltpu.VMEM((DCT, DCH), jnp.int32),
        pltpu.VMEM((DCH, 16), jnp.float32),
        pltpu.SemaphoreType.DMA,
        pltpu.SemaphoreType.DMA,
    ],
)


def _agg_body(xw_hbm, srcg_hbm, dsts_hbm, znd_hbm,
              out_hbm, acc, sidx, didx, bufs, semg, sems):
    c = lax.axis_index("c")
    s = lax.axis_index("s")
    sl = pl.ds(s * RPT, RPT)
    pltpu.sync_copy(znd_hbm.at[sl], acc.at[sl])
    plsc.subcore_barrier()

    lo = c * F0
    hi = F0 + c * (PHT - F0)

    @pl.loop(lo, hi)
    def _(p):
        pltpu.sync_copy(srcg_hbm.at[s, p], sidx)
        pltpu.sync_copy(dsts_hbm.at[s, p], didx)
        pltpu.async_copy(xw_hbm.at[sidx.at[0]], bufs[0], semg[0])
        pltpu.async_copy(xw_hbm.at[sidx.at[1]], bufs[1], semg[1])

        @pl.loop(0, CPC, step=4)
        def _(j):
            for b in range(4):
                ch = j + b
                buf = bufs[b]
                pltpu.make_async_copy(xw_hbm.at[sidx.at[ch]], buf, semg[b]).wait()
                pltpu.async_copy(buf, acc.at[didx.at[ch]], sems[b], add=True)

                b2 = (b + 2) % 4

                @pl.when(ch >= 2)
                def _():
                    pltpu.make_async_copy(
                        bufs[b2], acc.at[didx.at[0]], sems[b2]).wait()

                @pl.when(ch + 2 < CPC)
                def _():
                    pltpu.async_copy(xw_hbm.at[sidx.at[ch + 2]],
                                     bufs[b2], semg[b2])

        for b in (2, 3):
            pltpu.make_async_copy(bufs[b], acc.at[didx.at[0]], sems[b]).wait()

    plsc.subcore_barrier()
    pltpu.sync_copy(acc.at[sl], out_hbm.at[c, sl])


_agg_call = pl.kernel(
    _agg_body,
    out_type=jax.ShapeDtypeStruct((NCORES, NP, D), jnp.float32),
    mesh=_MESH,
    scratch_types=[
        pltpu.VMEM_SHARED((NP, D), jnp.float32),
        pltpu.VMEM((CPC, CH), jnp.int32),
        pltpu.VMEM((CPC, CH), jnp.int32),
        [pltpu.VMEM((CH, D), jnp.float32)] * 4,
        [pltpu.SemaphoreType.DMA] * 4,
        [pltpu.SemaphoreType.DMA] * 4,
    ],
)


def _tc_pre_body(dop_ref, dip_ref, h_ref, sn_ref, w_ref, norms_ref, xw_ref):
    dop = dop_ref[...]
    dip = dip_ref[...]
    deg_o = dop[0, :N, 0:1] + dop[1, :N, 0:1]
    deg_i = dip[0, :N, 0:1] + dip[1, :N, 0:1]
    ns = lax.rsqrt(jnp.maximum(deg_o, 1.0))
    nd = lax.rsqrt(jnp.maximum(deg_i, 1.0))
    sn = sn_ref[...]
    norms_ref[...] = jnp.concatenate([ns, nd, sn], axis=1)
    xw_ref[...] = jnp.dot(h_ref[...] * ns, w_ref[...],
                          preferred_element_type=jnp.float32)


def _tc_pre(dop, dip, h, sn, w):
    return pl.pallas_call(
        _tc_pre_body,
        out_shape=[jax.ShapeDtypeStruct((N, 3), jnp.float32),
                   jax.ShapeDtypeStruct((N, D), jnp.float32)],
    )(dop, dip, h, sn, w)


def _post_agg(h, aggp, norms, b, g, bb):
    agg = aggp[0, :N] + aggp[1, :N]
    nd = norms[:, 1:2]
    sn = norms[:, 2:3]
    hh = (agg * nd + b) * sn
    mu = jnp.mean(hh, axis=0, keepdims=True)
    cen = hh - mu
    var = jnp.mean(cen * cen, axis=0, keepdims=True)
    hn = cen * lax.rsqrt(var + EPS) * g + bb
    return h + jnp.maximum(hn, 0.0)


def _tc_mid_body(h_ref, ap_ref, norms_ref, b_ref, g_ref, bb_ref, w_ref,
                 hn_ref, xw_ref):
    norms = norms_ref[...]
    h_next = _post_agg(h_ref[...], ap_ref[...], norms,
                       b_ref[...], g_ref[...], bb_ref[...])
    hn_ref[...] = h_next
    xw_ref[...] = jnp.dot(h_next * norms[:, 0:1], w_ref[...],
                          preferred_element_type=jnp.float32)


def _tc_mid(h, aggp, norms, b, g, bb, w_next):
    return pl.pallas_call(
        _tc_mid_body,
        out_shape=[jax.ShapeDtypeStruct((N, D), jnp.float32),
                   jax.ShapeDtypeStruct((N, D), jnp.float32)],
    )(h, aggp, norms, b, g, bb, w_next)


def _tc_post_body(h_ref, ap_ref, norms_ref, b_ref, g_ref, bb_ref,
                  w0_ref, b0_ref, w1_ref, b1_ref, w2_ref, b2_ref, y_ref):
    h_next = _post_agg(h_ref[...], ap_ref[...], norms_ref[...],
                       b_ref[...], g_ref[...], bb_ref[...])
    hg = jnp.mean(h_next, axis=0, keepdims=True)
    y = jnp.maximum(jnp.dot(hg, w0_ref[...],
                            preferred_element_type=jnp.float32) + b0_ref[...], 0.0)
    y = jnp.maximum(jnp.dot(y, w1_ref[...],
                            preferred_element_type=jnp.float32) + b1_ref[...], 0.0)
    y_ref[...] = jnp.dot(y, w2_ref[...],
                         preferred_element_type=jnp.float32) + b2_ref[...]


def _tc_post(h, aggp, norms, b, g, bb, mw0, mb0, mw1, mb1, mw2, mb2):
    return pl.pallas_call(
        _tc_post_body,
        out_shape=jax.ShapeDtypeStruct((1, 10), jnp.float32),
    )(h, aggp, norms, b, g, bb, mw0, mb0, mw1, mb1, mw2, mb2)


def kernel(edge_index, nodes_feat, edges_feat, nodes_num_norm_sqrt,
           edges_num_norm_sqrt, params):
    del edges_feat, edges_num_norm_sqrt
    src = edge_index[0]
    dst = edge_index[1]
    pad = E_PAD - E
    pad0 = jnp.zeros((pad,), jnp.int32)
    padd = jnp.full((pad,), PAD_ROW, jnp.int32)
    shape4 = (NSUB, PHT, CPC, CH)
    shape_d = (NSUB, DCT, DCH)
    src_g = jnp.concatenate([src, pad0]).reshape(shape4)
    src_s = jnp.concatenate([src, padd]).reshape(shape_d)
    dst_s = jnp.concatenate([dst, padd]).reshape(shape4)
    zeros_nd = jnp.zeros((NP, D), jnp.float32)
    zeros_dg = jnp.zeros((NP, 16), jnp.float32)
    ones_ch = jnp.ones((DCH, 16), jnp.float32)

    dop, dip = _deg_call(src_s, dst_s.reshape(shape_d), zeros_dg, ones_ch)
    norms, xw = _tc_pre(dop, dip, nodes_feat, nodes_num_norm_sqrt,
                        params['W0'])

    def row(v):
        return v.reshape(1, -1)

    h = nodes_feat
    y = None
    for l in range(4):
        aggp = _agg_call(xw, src_g, dst_s, zeros_nd)
        b = row(params['b%d' % l])
        g = row(params['bn_g%d' % l])
        bb = row(params['bn_b%d' % l])
        if l < 3:
            h, xw = _tc_mid(h, aggp, norms, b, g, bb, params['W%d' % (l + 1)])
        else:
            y = _tc_post(h, aggp, norms, b, g, bb,
                         params['mlp_W0'], row(params['mlp_b0']),
                         params['mlp_W1'], row(params['mlp_b1']),
                         params['mlp_W2'], row(params['mlp_b2']))
    return y


# speedup vs baseline: 1.4162x; 1.4162x over previous
import functools

import jax
import jax.numpy as jnp
from jax import lax
from jax.experimental import pallas as pl
from jax.experimental.pallas import tpu as pltpu
from jax.experimental.pallas import tpu_sc as plsc

N = 10000
E = 320000
D = 128
EPS = 1e-5

NCORES = 2
NSUB = 16
CH = 64
CPC = 40
PHT = 8
F0 = 6
DCH = 128
DCT = PHT * CPC * CH // DCH
E_PAD = NSUB * PHT * CPC * CH
NP = 10240
RPT = NP // NSUB
PAD_ROW = 10008

_MESH = plsc.VectorSubcoreMesh(
    core_axis_name="c", subcore_axis_name="s",
    num_cores=NCORES, num_subcores=NSUB)


def _deg_body(srcs_hbm, dsts_hbm, zdeg_hbm, ones_hbm,
              do_hbm, di_hbm, h_out, h_in, sidx, didx, ones_v, sem_o, sem_i):
    c = lax.axis_index("c")
    s = lax.axis_index("s")
    pltpu.sync_copy(srcs_hbm.at[s], sidx)
    pltpu.sync_copy(dsts_hbm.at[s], didx)
    pltpu.sync_copy(ones_hbm, ones_v)
    sl = pl.ds(s * RPT, RPT)
    pltpu.sync_copy(zdeg_hbm.at[sl], h_out.at[sl])
    pltpu.sync_copy(zdeg_hbm.at[sl], h_in.at[sl])
    plsc.subcore_barrier()

    LAG = 4
    lo = c * (DCT // NCORES)
    hi = lo + DCT // NCORES

    @pl.loop(lo, hi)
    def _(j):
        pltpu.async_copy(ones_v, h_out.at[sidx.at[j]], sem_o, add=True)
        pltpu.async_copy(ones_v, h_in.at[didx.at[j]], sem_i, add=True)

        @pl.when(j >= lo + LAG)
        def _():
            pltpu.make_async_copy(ones_v, h_out.at[sidx.at[0]], sem_o).wait()
            pltpu.make_async_copy(ones_v, h_in.at[didx.at[0]], sem_i).wait()

    for _ in range(LAG):
        pltpu.make_async_copy(ones_v, h_out.at[sidx.at[0]], sem_o).wait()
        pltpu.make_async_copy(ones_v, h_in.at[didx.at[0]], sem_i).wait()

    plsc.subcore_barrier()
    pltpu.sync_copy(h_out.at[sl], do_hbm.at[c, sl])
    pltpu.sync_copy(h_in.at[sl], di_hbm.at[c, sl])


_deg_call = pl.kernel(
    _deg_body,
    out_type=[jax.ShapeDtypeStruct((NCORES, NP, 16), jnp.float32),
              jax.ShapeDtypeStruct((NCORES, NP, 16), jnp.float32)],
    mesh=_MESH,
    scratch_types=[
        pltpu.VMEM_SHARED((NP, 16), jnp.float32),
        pltpu.VMEM_SHARED((NP, 16), jnp.float32),
        pltpu.VMEM((DCT, DCH), jnp.int32),
        pltpu.VMEM((DCT, DCH), jnp.int32),
        pltpu.VMEM((DCH, 16), jnp.float32),
        pltpu.SemaphoreType.DMA,
        pltpu.SemaphoreType.DMA,
    ],
)


def _agg_body(xw_hbm, srcg_hbm, dsts_hbm, znd_hbm,
              out_hbm, acc, sidx, didx, bufs, semg, sems):
    c = lax.axis_index("c")
    s = lax.axis_index("s")
    sl = pl.ds(s * RPT, RPT)
    pltpu.sync_copy(znd_hbm.at[sl], acc.at[sl])
    plsc.subcore_barrier()

    lo = c * F0
    hi = F0 + c * (PHT - F0)

    @pl.loop(lo, hi)
    def _(p):
        pltpu.sync_copy(srcg_hbm.at[s, p], sidx)
        pltpu.sync_copy(dsts_hbm.at[s, p], didx)
        pltpu.async_copy(xw_hbm.at[sidx.at[0]], bufs[0], semg[0])
        pltpu.async_copy(xw_hbm.at[sidx.at[1]], bufs[1], semg[1])

        @pl.loop(0, CPC, step=4)
        def _(j):
            for b in range(4):
                ch = j + b
                buf = bufs[b]
                pltpu.make_async_copy(xw_hbm.at[sidx.at[ch]], buf, semg[b]).wait()
                pltpu.async_copy(buf, acc.at[didx.at[ch]], sems[b], add=True)

                b2 = (b + 2) % 4

                @pl.when(ch >= 2)
                def _():
                    pltpu.make_async_copy(
                        bufs[b2], acc.at[didx.at[0]], sems[b2]).wait()

                @pl.when(ch + 2 < CPC)
                def _():
                    pltpu.async_copy(xw_hbm.at[sidx.at[ch + 2]],
                                     bufs[b2], semg[b2])

        for b in (2, 3):
            pltpu.make_async_copy(bufs[b], acc.at[didx.at[0]], sems[b]).wait()

    plsc.subcore_barrier()
    pltpu.sync_copy(acc.at[sl], out_hbm.at[c, sl])


_agg_call = pl.kernel(
    _agg_body,
    out_type=jax.ShapeDtypeStruct((NCORES, NP, D), jnp.float32),
    mesh=_MESH,
    scratch_types=[
        pltpu.VMEM_SHARED((NP, D), jnp.float32),
        pltpu.VMEM((CPC, CH), jnp.int32),
        pltpu.VMEM((CPC, CH), jnp.int32),
        [pltpu.VMEM((CH, D), jnp.float32)] * 4,
        [pltpu.SemaphoreType.DMA] * 4,
        [pltpu.SemaphoreType.DMA] * 4,
    ],
)


def _tc_pre_body(dop_ref, dip_ref, h_ref, sn_ref, w_ref, norms_ref, xw_ref):
    dop = dop_ref[...]
    dip = dip_ref[...]
    deg_o = dop[0, :N, 0:1] + dop[1, :N, 0:1]
    deg_i = dip[0, :N, 0:1] + dip[1, :N, 0:1]
    ns = lax.rsqrt(jnp.maximum(deg_o, 1.0))
    nd = lax.rsqrt(jnp.maximum(deg_i, 1.0))
    sn = sn_ref[...]
    norms_ref[...] = jnp.concatenate([ns, nd, sn], axis=1)
    xw_ref[...] = jnp.dot(h_ref[...] * ns, w_ref[...],
                          preferred_element_type=jnp.float32)


def _tc_pre(dop, dip, h, sn, w):
    return pl.pallas_call(
        _tc_pre_body,
        out_shape=[jax.ShapeDtypeStruct((N, 3), jnp.float32),
                   jax.ShapeDtypeStruct((N, D), jnp.float32)],
    )(dop, dip, h, sn, w)


def _post_agg(h, aggp, norms, b, g, bb):
    agg = aggp[0, :N] + aggp[1, :N]
    nd = norms[:, 1:2]
    sn = norms[:, 2:3]
    hh = (agg * nd + b) * sn
    mu = jnp.mean(hh, axis=0, keepdims=True)
    cen = hh - mu
    var = jnp.mean(cen * cen, axis=0, keepdims=True)
    hn = cen * lax.rsqrt(var + EPS) * g + bb
    return h + jnp.maximum(hn, 0.0)


def _tc_mid_body(h_ref, ap_ref, norms_ref, b_ref, g_ref, bb_ref, w_ref,
                 hn_ref, xw_ref):
    norms = norms_ref[...]
    h_next = _post_agg(h_ref[...], ap_ref[...], norms,
                       b_ref[...], g_ref[...], bb_ref[...])
    hn_ref[...] = h_next
    xw_ref[...] = jnp.dot(h_next * norms[:, 0:1], w_ref[...],
                          preferred_element_type=jnp.float32)


def _tc_mid(h, aggp, norms, b, g, bb, w_next):
    return pl.pallas_call(
        _tc_mid_body,
        out_shape=[jax.ShapeDtypeStruct((N, D), jnp.float32),
                   jax.ShapeDtypeStruct((N, D), jnp.float32)],
    )(h, aggp, norms, b, g, bb, w_next)


def _tc_post_body(h_ref, ap_ref, norms_ref, b_ref, g_ref, bb_ref,
                  w0_ref, b0_ref, w1_ref, b1_ref, w2_ref, b2_ref, y_ref):
    h_next = _post_agg(h_ref[...], ap_ref[...], norms_ref[...],
                       b_ref[...], g_ref[...], bb_ref[...])
    hg = jnp.mean(h_next, axis=0, keepdims=True)
    y = jnp.maximum(jnp.dot(hg, w0_ref[...],
                            preferred_element_type=jnp.float32) + b0_ref[...], 0.0)
    y = jnp.maximum(jnp.dot(y, w1_ref[...],
                            preferred_element_type=jnp.float32) + b1_ref[...], 0.0)
    y_ref[...] = jnp.dot(y, w2_ref[...],
                         preferred_element_type=jnp.float32) + b2_ref[...]


def _tc_post(h, aggp, norms, b, g, bb, mw0, mb0, mw1, mb1, mw2, mb2):
    return pl.pallas_call(
        _tc_post_body,
        out_shape=jax.ShapeDtypeStruct((1, 10), jnp.float32),
    )(h, aggp, norms, b, g, bb, mw0, mb0, mw1, mb1, mw2, mb2)


def kernel(edge_index, nodes_feat, edges_feat, nodes_num_norm_sqrt,
           edges_num_norm_sqrt, params):
    del edges_feat, edges_num_norm_sqrt
    src = edge_index[0]
    dst = edge_index[1]
    pad = E_PAD - E
    pad0 = jnp.zeros((pad,), jnp.int32)
    padd = jnp.full((pad,), PAD_ROW, jnp.int32)
    shape4 = (NSUB, PHT, CPC, CH)
    shape_d = (NSUB, DCT, DCH)
    src_g = jnp.concatenate([src, pad0]).reshape(shape4)
    src_s = jnp.concatenate([src, padd]).reshape(shape_d)
    dst_s = jnp.concatenate([dst, padd]).reshape(shape4)
    zeros_nd = jnp.zeros((NP, D), jnp.float32)
    zeros_dg = jnp.zeros((NP, 16), jnp.float32)
    ones_ch = jnp.ones((DCH, 16), jnp.float32)

    dop, dip = _deg_call(src_s, dst_s.reshape(shape_d), zeros_dg, ones_ch)
    norms, xw = _tc_pre(dop, dip, nodes_feat, nodes_num_norm_sqrt,
                        params['W0'])

    def row(v):
        return v.reshape(1, -1)

    h = nodes_feat
    y = None
    for l in range(4):
        aggp = _agg_call(xw, src_g, dst_s, zeros_nd)
        b = row(params['b%d' % l])
        g = row(params['bn_g%d' % l])
        bb = row(params['bn_b%d' % l])
        if l < 3:
            h, xw = _tc_mid(h, aggp, norms, b, g, bb, params['W%d' % (l + 1)])
        else:
            y = _tc_post(h, aggp, norms, b, g, bb,
                         params['mlp_W0'], row(params['mlp_b0']),
                         params['mlp_W1'], row(params['mlp_b1']),
                         params['mlp_W2'], row(params['mlp_b2']))
    return y


# speedup vs baseline: 2.9272x; 2.0670x over previous
import functools

import jax
import jax.numpy as jnp
from jax import lax
from jax.experimental import pallas as pl
from jax.experimental.pallas import tpu as pltpu
from jax.experimental.pallas import tpu_sc as plsc

N = 10000
E = 320000
D = 128
EPS = 1e-5

NCORES = 2
NSUB = 16
CH = 64
CPC = 40
PHT = 8
F0 = 4
DCH = 128
DCT = PHT * CPC * CH // DCH
E_PAD = NSUB * PHT * CPC * CH
NP = 10240
RPT = NP // NSUB
PAD_ROW = 10008

_MESH = plsc.VectorSubcoreMesh(
    core_axis_name="c", subcore_axis_name="s",
    num_cores=NCORES, num_subcores=NSUB)


def _deg_body(srcs_hbm, dsts_hbm, zdeg_hbm, ones_hbm,
              do_hbm, di_hbm, h_out, h_in, sidx, didx, ones_v, sem_o, sem_i):
    c = lax.axis_index("c")
    s = lax.axis_index("s")
    pltpu.sync_copy(srcs_hbm.at[s], sidx)
    pltpu.sync_copy(dsts_hbm.at[s], didx)
    pltpu.sync_copy(ones_hbm, ones_v)
    sl = pl.ds(s * RPT, RPT)
    pltpu.sync_copy(zdeg_hbm.at[sl], h_out.at[sl])
    pltpu.sync_copy(zdeg_hbm.at[sl], h_in.at[sl])
    plsc.subcore_barrier()

    LAG = 4
    lo = c * (DCT // NCORES)
    hi = lo + DCT // NCORES

    @pl.loop(lo, hi)
    def _(j):
        pltpu.async_copy(ones_v, h_out.at[sidx.at[j]], sem_o, add=True)
        pltpu.async_copy(ones_v, h_in.at[didx.at[j]], sem_i, add=True)

        @pl.when(j >= lo + LAG)
        def _():
            pltpu.make_async_copy(ones_v, h_out.at[sidx.at[0]], sem_o).wait()
            pltpu.make_async_copy(ones_v, h_in.at[didx.at[0]], sem_i).wait()

    for _ in range(LAG):
        pltpu.make_async_copy(ones_v, h_out.at[sidx.at[0]], sem_o).wait()
        pltpu.make_async_copy(ones_v, h_in.at[didx.at[0]], sem_i).wait()

    plsc.subcore_barrier()
    pltpu.sync_copy(h_out.at[sl], do_hbm.at[c, sl])
    pltpu.sync_copy(h_in.at[sl], di_hbm.at[c, sl])


_deg_call = pl.kernel(
    _deg_body,
    out_type=[jax.ShapeDtypeStruct((NCORES, NP, 16), jnp.float32),
              jax.ShapeDtypeStruct((NCORES, NP, 16), jnp.float32)],
    mesh=_MESH,
    scratch_types=[
        pltpu.VMEM_SHARED((NP, 16), jnp.float32),
        pltpu.VMEM_SHARED((NP, 16), jnp.float32),
        pltpu.VMEM((DCT, DCH), jnp.int32),
        pltpu.VMEM((DCT, DCH), jnp.int32),
        pltpu.VMEM((DCH, 16), jnp.float32),
        pltpu.SemaphoreType.DMA,
        pltpu.SemaphoreType.DMA,
    ],
)


def _agg_body(xw_hbm, srcg_hbm, dsts_hbm, znd_hbm,
              out_hbm, acc, sidx, didx, bufs, semg, sems):
    c = lax.axis_index("c")
    s = lax.axis_index("s")
    sl = pl.ds(s * RPT, RPT)
    pltpu.sync_copy(znd_hbm.at[sl], acc.at[sl])
    plsc.subcore_barrier()

    lo = c * F0
    hi = F0 + c * (PHT - F0)

    @pl.loop(lo, hi)
    def _(p):
        pltpu.sync_copy(srcg_hbm.at[s, p], sidx)
        pltpu.sync_copy(dsts_hbm.at[s, p], didx)
        pltpu.async_copy(xw_hbm.at[sidx.at[0]], bufs[0], semg[0])
        pltpu.async_copy(xw_hbm.at[sidx.at[1]], bufs[1], semg[1])

        @pl.loop(0, CPC, step=4)
        def _(j):
            for b in range(4):
                ch = j + b
                buf = bufs[b]
                pltpu.make_async_copy(xw_hbm.at[sidx.at[ch]], buf, semg[b]).wait()
                pltpu.async_copy(buf, acc.at[didx.at[ch]], sems[b], add=True)

                b2 = (b + 2) % 4

                @pl.when(ch >= 2)
                def _():
                    pltpu.make_async_copy(
                        bufs[b2], acc.at[didx.at[0]], sems[b2]).wait()

                @pl.when(ch + 2 < CPC)
                def _():
                    pltpu.async_copy(xw_hbm.at[sidx.at[ch + 2]],
                                     bufs[b2], semg[b2])

        for b in (2, 3):
            pltpu.make_async_copy(bufs[b], acc.at[didx.at[0]], sems[b]).wait()

    plsc.subcore_barrier()
    pltpu.sync_copy(acc.at[sl], out_hbm.at[c, sl])


_agg_call = pl.kernel(
    _agg_body,
    out_type=jax.ShapeDtypeStruct((NCORES, NP, D), jnp.float32),
    mesh=_MESH,
    scratch_types=[
        pltpu.VMEM_SHARED((NP, D), jnp.float32),
        pltpu.VMEM((CPC, CH), jnp.int32),
        pltpu.VMEM((CPC, CH), jnp.int32),
        [pltpu.VMEM((CH, D), jnp.float32)] * 4,
        [pltpu.SemaphoreType.DMA] * 4,
        [pltpu.SemaphoreType.DMA] * 4,
    ],
)


def _tc_pre_body(dop_ref, dip_ref, h_ref, sn_ref, w_ref, norms_ref, xw_ref):
    dop = dop_ref[...]
    dip = dip_ref[...]
    deg_o = dop[0, :N, 0:1] + dop[1, :N, 0:1]
    deg_i = dip[0, :N, 0:1] + dip[1, :N, 0:1]
    ns = lax.rsqrt(jnp.maximum(deg_o, 1.0))
    nd = lax.rsqrt(jnp.maximum(deg_i, 1.0))
    sn = sn_ref[...]
    norms_ref[...] = jnp.concatenate([ns, nd, sn], axis=1)
    xw_ref[...] = jnp.dot(h_ref[...] * ns, w_ref[...],
                          preferred_element_type=jnp.float32)


def _tc_pre(dop, dip, h, sn, w):
    return pl.pallas_call(
        _tc_pre_body,
        out_shape=[jax.ShapeDtypeStruct((N, 3), jnp.float32),
                   jax.ShapeDtypeStruct((N, D), jnp.float32)],
    )(dop, dip, h, sn, w)


def _post_agg(h, aggp, norms, b, g, bb):
    agg = aggp[0, :N] + aggp[1, :N]
    nd = norms[:, 1:2]
    sn = norms[:, 2:3]
    hh = (agg * nd + b) * sn
    mu = jnp.mean(hh, axis=0, keepdims=True)
    cen = hh - mu
    var = jnp.mean(cen * cen, axis=0, keepdims=True)
    hn = cen * lax.rsqrt(var + EPS) * g + bb
    return h + jnp.maximum(hn, 0.0)


def _tc_mid_body(h_ref, ap_ref, norms_ref, b_ref, g_ref, bb_ref, w_ref,
                 hn_ref, xw_ref):
    norms = norms_ref[...]
    h_next = _post_agg(h_ref[...], ap_ref[...], norms,
                       b_ref[...], g_ref[...], bb_ref[...])
    hn_ref[...] = h_next
    xw_ref[...] = jnp.dot(h_next * norms[:, 0:1], w_ref[...],
                          preferred_element_type=jnp.float32)


def _tc_mid(h, aggp, norms, b, g, bb, w_next):
    return pl.pallas_call(
        _tc_mid_body,
        out_shape=[jax.ShapeDtypeStruct((N, D), jnp.float32),
                   jax.ShapeDtypeStruct((N, D), jnp.float32)],
    )(h, aggp, norms, b, g, bb, w_next)


def _tc_post_body(h_ref, ap_ref, norms_ref, b_ref, g_ref, bb_ref,
                  w0_ref, b0_ref, w1_ref, b1_ref, w2_ref, b2_ref, y_ref):
    h_next = _post_agg(h_ref[...], ap_ref[...], norms_ref[...],
                       b_ref[...], g_ref[...], bb_ref[...])
    hg = jnp.mean(h_next, axis=0, keepdims=True)
    y = jnp.maximum(jnp.dot(hg, w0_ref[...],
                            preferred_element_type=jnp.float32) + b0_ref[...], 0.0)
    y = jnp.maximum(jnp.dot(y, w1_ref[...],
                            preferred_element_type=jnp.float32) + b1_ref[...], 0.0)
    y_ref[...] = jnp.dot(y, w2_ref[...],
                         preferred_element_type=jnp.float32) + b2_ref[...]


def _tc_post(h, aggp, norms, b, g, bb, mw0, mb0, mw1, mb1, mw2, mb2):
    return pl.pallas_call(
        _tc_post_body,
        out_shape=jax.ShapeDtypeStruct((1, 10), jnp.float32),
    )(h, aggp, norms, b, g, bb, mw0, mb0, mw1, mb1, mw2, mb2)


def kernel(edge_index, nodes_feat, edges_feat, nodes_num_norm_sqrt,
           edges_num_norm_sqrt, params):
    del edges_feat, edges_num_norm_sqrt
    src = edge_index[0]
    dst = edge_index[1]
    pad = E_PAD - E
    pad_i = jnp.arange(pad, dtype=jnp.int32)
    pad0 = pad_i % N
    padd = N + pad_i % (NP - N)
    shape4 = (NSUB, PHT, CPC, CH)
    shape_d = (NSUB, DCT, DCH)
    src_g = jnp.concatenate([src, pad0]).reshape(shape4)
    src_s = jnp.concatenate([src, padd]).reshape(shape_d)
    dst_s = jnp.concatenate([dst, padd]).reshape(shape4)
    zeros_nd = jnp.zeros((NP, D), jnp.float32)
    zeros_dg = jnp.zeros((NP, 16), jnp.float32)
    ones_ch = jnp.ones((DCH, 16), jnp.float32)

    dop, dip = _deg_call(src_s, dst_s.reshape(shape_d), zeros_dg, ones_ch)
    norms, xw = _tc_pre(dop, dip, nodes_feat, nodes_num_norm_sqrt,
                        params['W0'])

    def row(v):
        return v.reshape(1, -1)

    h = nodes_feat
    y = None
    for l in range(4):
        aggp = _agg_call(xw, src_g, dst_s, zeros_nd)
        b = row(params['b%d' % l])
        g = row(params['bn_g%d' % l])
        bb = row(params['bn_b%d' % l])
        if l < 3:
            h, xw = _tc_mid(h, aggp, norms, b, g, bb, params['W%d' % (l + 1)])
        else:
            y = _tc_post(h, aggp, norms, b, g, bb,
                         params['mlp_W0'], row(params['mlp_b0']),
                         params['mlp_W1'], row(params['mlp_b1']),
                         params['mlp_W2'], row(params['mlp_b2']))
    return y


# speedup vs baseline: 2.9296x; 1.0008x over previous
import functools

import jax
import jax.numpy as jnp
from jax import lax
from jax.experimental import pallas as pl
from jax.experimental.pallas import tpu as pltpu
from jax.experimental.pallas import tpu_sc as plsc

N = 10000
E = 320000
D = 128
EPS = 1e-5

NCORES = 2
NSUB = 16
CH = 64
CPC = 40
PHT = 8
F0 = 4
DCH = 128
DCT = PHT * CPC * CH // DCH
E_PAD = NSUB * PHT * CPC * CH
NP = 10240
RPT = NP // NSUB
PAD_ROW = 10008

_MESH = plsc.VectorSubcoreMesh(
    core_axis_name="c", subcore_axis_name="s",
    num_cores=NCORES, num_subcores=NSUB)


def _deg_body(srcs_hbm, dsts_hbm, zdeg_hbm, ones_hbm,
              do_hbm, di_hbm, h_out, h_in, sidx, didx, ones_v, sem_o, sem_i):
    c = lax.axis_index("c")
    s = lax.axis_index("s")
    pltpu.sync_copy(srcs_hbm.at[s], sidx)
    pltpu.sync_copy(dsts_hbm.at[s], didx)
    pltpu.sync_copy(ones_hbm, ones_v)
    sl = pl.ds(s * RPT, RPT)
    pltpu.sync_copy(zdeg_hbm.at[sl], h_out.at[sl])
    pltpu.sync_copy(zdeg_hbm.at[sl], h_in.at[sl])
    plsc.subcore_barrier()

    LAG = 8
    lo = c * (DCT // NCORES)
    hi = lo + DCT // NCORES

    @pl.loop(lo, hi)
    def _(j):
        pltpu.async_copy(ones_v, h_out.at[sidx.at[j]], sem_o, add=True)
        pltpu.async_copy(ones_v, h_in.at[didx.at[j]], sem_i, add=True)

        @pl.when(j >= lo + LAG)
        def _():
            pltpu.make_async_copy(ones_v, h_out.at[sidx.at[0]], sem_o).wait()
            pltpu.make_async_copy(ones_v, h_in.at[didx.at[0]], sem_i).wait()

    for _ in range(LAG):
        pltpu.make_async_copy(ones_v, h_out.at[sidx.at[0]], sem_o).wait()
        pltpu.make_async_copy(ones_v, h_in.at[didx.at[0]], sem_i).wait()

    plsc.subcore_barrier()
    pltpu.sync_copy(h_out.at[sl], do_hbm.at[c, sl])
    pltpu.sync_copy(h_in.at[sl], di_hbm.at[c, sl])


_deg_call = pl.kernel(
    _deg_body,
    out_type=[jax.ShapeDtypeStruct((NCORES, NP, 16), jnp.float32),
              jax.ShapeDtypeStruct((NCORES, NP, 16), jnp.float32)],
    mesh=_MESH,
    scratch_types=[
        pltpu.VMEM_SHARED((NP, 16), jnp.float32),
        pltpu.VMEM_SHARED((NP, 16), jnp.float32),
        pltpu.VMEM((DCT, DCH), jnp.int32),
        pltpu.VMEM((DCT, DCH), jnp.int32),
        pltpu.VMEM((DCH, 16), jnp.float32),
        pltpu.SemaphoreType.DMA,
        pltpu.SemaphoreType.DMA,
    ],
)


def _agg_body(xw_hbm, srcg_hbm, dsts_hbm, znd_hbm,
              out_hbm, acc, sidx, didx, bufs, semg, sems):
    c = lax.axis_index("c")
    s = lax.axis_index("s")
    sl = pl.ds(s * RPT, RPT)
    pltpu.sync_copy(znd_hbm.at[sl], acc.at[sl])
    plsc.subcore_barrier()

    lo = c * F0
    hi = F0 + c * (PHT - F0)

    @pl.loop(lo, hi)
    def _(p):
        pltpu.sync_copy(srcg_hbm.at[s, p], sidx)
        pltpu.sync_copy(dsts_hbm.at[s, p], didx)
        pltpu.async_copy(xw_hbm.at[sidx.at[0]], bufs[0], semg[0])
        pltpu.async_copy(xw_hbm.at[sidx.at[1]], bufs[1], semg[1])

        @pl.loop(0, CPC, step=4)
        def _(j):
            for b in range(4):
                ch = j + b
                buf = bufs[b]
                pltpu.make_async_copy(xw_hbm.at[sidx.at[ch]], buf, semg[b]).wait()
                pltpu.async_copy(buf, acc.at[didx.at[ch]], sems[b], add=True)

                b2 = (b + 2) % 4

                @pl.when(ch >= 2)
                def _():
                    pltpu.make_async_copy(
                        bufs[b2], acc.at[didx.at[0]], sems[b2]).wait()

                @pl.when(ch + 2 < CPC)
                def _():
                    pltpu.async_copy(xw_hbm.at[sidx.at[ch + 2]],
                                     bufs[b2], semg[b2])

        for b in (2, 3):
            pltpu.make_async_copy(bufs[b], acc.at[didx.at[0]], sems[b]).wait()

    plsc.subcore_barrier()
    pltpu.sync_copy(acc.at[sl], out_hbm.at[c, sl])


_agg_call = pl.kernel(
    _agg_body,
    out_type=jax.ShapeDtypeStruct((NCORES, NP, D), jnp.float32),
    mesh=_MESH,
    scratch_types=[
        pltpu.VMEM_SHARED((NP, D), jnp.float32),
        pltpu.VMEM((CPC, CH), jnp.int32),
        pltpu.VMEM((CPC, CH), jnp.int32),
        [pltpu.VMEM((CH, D), jnp.float32)] * 4,
        [pltpu.SemaphoreType.DMA] * 4,
        [pltpu.SemaphoreType.DMA] * 4,
    ],
)


def _tc_pre_body(dop_ref, dip_ref, h_ref, sn_ref, w_ref, norms_ref, xw_ref):
    dop = dop_ref[...]
    dip = dip_ref[...]
    deg_o = dop[0, :N, 0:1] + dop[1, :N, 0:1]
    deg_i = dip[0, :N, 0:1] + dip[1, :N, 0:1]
    ns = lax.rsqrt(jnp.maximum(deg_o, 1.0))
    nd = lax.rsqrt(jnp.maximum(deg_i, 1.0))
    sn = sn_ref[...]
    norms_ref[...] = jnp.concatenate([ns, nd, sn], axis=1)
    xw_ref[...] = jnp.dot(h_ref[...] * ns, w_ref[...],
                          preferred_element_type=jnp.float32)


def _tc_pre(dop, dip, h, sn, w):
    return pl.pallas_call(
        _tc_pre_body,
        out_shape=[jax.ShapeDtypeStruct((N, 3), jnp.float32),
                   jax.ShapeDtypeStruct((N, D), jnp.float32)],
    )(dop, dip, h, sn, w)


def _post_agg(h, aggp, norms, b, g, bb):
    agg = aggp[0, :N] + aggp[1, :N]
    nd = norms[:, 1:2]
    sn = norms[:, 2:3]
    hh = (agg * nd + b) * sn
    mu = jnp.mean(hh, axis=0, keepdims=True)
    cen = hh - mu
    var = jnp.mean(cen * cen, axis=0, keepdims=True)
    hn = cen * lax.rsqrt(var + EPS) * g + bb
    return h + jnp.maximum(hn, 0.0)


def _tc_mid_body(h_ref, ap_ref, norms_ref, b_ref, g_ref, bb_ref, w_ref,
                 hn_ref, xw_ref):
    norms = norms_ref[...]
    h_next = _post_agg(h_ref[...], ap_ref[...], norms,
                       b_ref[...], g_ref[...], bb_ref[...])
    hn_ref[...] = h_next
    xw_ref[...] = jnp.dot(h_next * norms[:, 0:1], w_ref[...],
                          preferred_element_type=jnp.float32)


def _tc_mid(h, aggp, norms, b, g, bb, w_next):
    return pl.pallas_call(
        _tc_mid_body,
        out_shape=[jax.ShapeDtypeStruct((N, D), jnp.float32),
                   jax.ShapeDtypeStruct((N, D), jnp.float32)],
    )(h, aggp, norms, b, g, bb, w_next)


def _tc_post_body(h_ref, ap_ref, norms_ref, b_ref, g_ref, bb_ref,
                  w0_ref, b0_ref, w1_ref, b1_ref, w2_ref, b2_ref, y_ref):
    h_next = _post_agg(h_ref[...], ap_ref[...], norms_ref[...],
                       b_ref[...], g_ref[...], bb_ref[...])
    hg = jnp.mean(h_next, axis=0, keepdims=True)
    y = jnp.maximum(jnp.dot(hg, w0_ref[...],
                            preferred_element_type=jnp.float32) + b0_ref[...], 0.0)
    y = jnp.maximum(jnp.dot(y, w1_ref[...],
                            preferred_element_type=jnp.float32) + b1_ref[...], 0.0)
    y_ref[...] = jnp.dot(y, w2_ref[...],
                         preferred_element_type=jnp.float32) + b2_ref[...]


def _tc_post(h, aggp, norms, b, g, bb, mw0, mb0, mw1, mb1, mw2, mb2):
    return pl.pallas_call(
        _tc_post_body,
        out_shape=jax.ShapeDtypeStruct((1, 10), jnp.float32),
    )(h, aggp, norms, b, g, bb, mw0, mb0, mw1, mb1, mw2, mb2)


def kernel(edge_index, nodes_feat, edges_feat, nodes_num_norm_sqrt,
           edges_num_norm_sqrt, params):
    del edges_feat, edges_num_norm_sqrt
    src = edge_index[0]
    dst = edge_index[1]
    pad = E_PAD - E
    pad_i = jnp.arange(pad, dtype=jnp.int32)
    pad0 = pad_i % N
    padd = N + pad_i % (NP - N)
    shape4 = (NSUB, PHT, CPC, CH)
    shape_d = (NSUB, DCT, DCH)
    src_g = jnp.concatenate([src, pad0]).reshape(shape4)
    src_s = jnp.concatenate([src, padd]).reshape(shape_d)
    dst_s = jnp.concatenate([dst, padd]).reshape(shape4)
    zeros_nd = jnp.zeros((NP, D), jnp.float32)
    zeros_dg = jnp.zeros((NP, 16), jnp.float32)
    ones_ch = jnp.ones((DCH, 16), jnp.float32)

    dop, dip = _deg_call(src_s, dst_s.reshape(shape_d), zeros_dg, ones_ch)
    norms, xw = _tc_pre(dop, dip, nodes_feat, nodes_num_norm_sqrt,
                        params['W0'])

    def row(v):
        return v.reshape(1, -1)

    h = nodes_feat
    y = None
    for l in range(4):
        aggp = _agg_call(xw, src_g, dst_s, zeros_nd)
        b = row(params['b%d' % l])
        g = row(params['bn_g%d' % l])
        bb = row(params['bn_b%d' % l])
        if l < 3:
            h, xw = _tc_mid(h, aggp, norms, b, g, bb, params['W%d' % (l + 1)])
        else:
            y = _tc_post(h, aggp, norms, b, g, bb,
                         params['mlp_W0'], row(params['mlp_b0']),
                         params['mlp_W1'], row(params['mlp_b1']),
                         params['mlp_W2'], row(params['mlp_b2']))
    return y
